# Initial kernel scaffold; baseline (speedup 1.0000x reference)
import jax
import jax.numpy as jnp
from jax.experimental import pallas as pl


def kernel(x, edge_index, batch, W1, b1, att1, W2, b2, att2, Wfc, bfc):
    raise NotImplementedError("write your pallas kernel here")


# speedup vs baseline: 2.0338x; 2.0338x over previous
import functools

import jax
import jax.numpy as jnp
from jax import lax
from jax.experimental import pallas as pl
from jax.experimental.pallas import tpu as pltpu
from jax.experimental.pallas import tpu_sc as plsc

_N = 10000
_D = 256
_HID = 256
_HEADS = 8
_NGRAPH = 8

_E_RAW = 160000
_E_FULL = _E_RAW + _N
_E_PAD = 172032
_EPT = _E_PAD // 16
_G = 64
_NGRP = _EPT // _G

_NP = 10240
_NT = 10016

_RB = 400
_NRB = _N // _RB


def _tc1_body(x_ref, w_ref, b_ref, a1d_ref, a1s_ref, hp_ref, ad_ref, as_ref):
  j = pl.program_id(1)
  hp = jnp.dot(x_ref[...], w_ref[...], preferred_element_type=jnp.float32)
  hp = hp + b_ref[...]
  hp_ref[...] = hp.reshape(1, _RB, 128)
  ad = jnp.dot(hp, a1d_ref[...], preferred_element_type=jnp.float32)
  asr = jnp.dot(hp, a1s_ref[...], preferred_element_type=jnp.float32)

  @pl.when(j == 0)
  def _init():
    ad_ref[...] = ad
    as_ref[...] = asr

  @pl.when(j > 0)
  def _acc():
    ad_ref[...] += ad
    as_ref[...] += asr


def _tc1(x, W1, b1, A1d, A1s):
  return pl.pallas_call(
      _tc1_body,
      grid=(_NRB, 16),
      in_specs=[
          pl.BlockSpec((_RB, _D), lambda i, j: (i, 0)),
          pl.BlockSpec((_D, 128), lambda i, j: (0, j)),
          pl.BlockSpec((1, 128), lambda i, j: (0, j)),
          pl.BlockSpec((128, 128), lambda i, j: (j, 0)),
          pl.BlockSpec((128, 128), lambda i, j: (j, 0)),
      ],
      out_specs=[
          pl.BlockSpec((1, _RB, 128), lambda i, j: (j, i, 0)),
          pl.BlockSpec((_RB, 128), lambda i, j: (i, 0)),
          pl.BlockSpec((_RB, 128), lambda i, j: (i, 0)),
      ],
      out_shape=[
          jax.ShapeDtypeStruct((16, _N, 128), jnp.float32),
          jax.ShapeDtypeStruct((_N, 128), jnp.float32),
          jax.ShapeDtypeStruct((_N, 128), jnp.float32),
      ],
  )(x, W1, b1, A1d, A1s)


def _tc2_body(acc_ref, den_ref, w_ref, b_ref, a2d_ref, a2s_ref,
              h2_ref, ad_ref, as_ref):
  j = pl.program_id(1)
  den = den_ref[0] + den_ref[1]
  cmask = lax.broadcasted_iota(jnp.int32, (_RB, 128), 1) == (j // 2)
  dcol = jnp.sum(jnp.where(cmask, den, 0.0), axis=1, keepdims=True)
  hin = (acc_ref[0, 0] + acc_ref[0, 1]) / (dcol + 1e-16)
  hin = jnp.where(hin > 0, hin, jnp.exp(hin) - 1.0)
  part = jnp.dot(hin, w_ref[...], preferred_element_type=jnp.float32)

  @pl.when(j == 0)
  def _init():
    h2_ref[...] = part + b_ref[...]

  @pl.when(j > 0)
  def _acc():
    h2_ref[...] += part

  @pl.when(j == 15)
  def _fin():
    h2 = h2_ref[...]
    ad_ref[...] = jnp.dot(h2, a2d_ref[...], preferred_element_type=jnp.float32)
    as_ref[...] = jnp.dot(h2, a2s_ref[...], preferred_element_type=jnp.float32)


def _tc2(acc1, den1, W2, b2, A2d, A2s):
  return pl.pallas_call(
      _tc2_body,
      grid=(_NRB, 16),
      in_specs=[
          pl.BlockSpec((1, 2, _RB, 128), lambda i, j: (j, 0, i, 0)),
          pl.BlockSpec((2, _RB, 128), lambda i, j: (0, i, 0)),
          pl.BlockSpec((128, _HID), lambda i, j: (j, 0)),
          pl.BlockSpec((1, _HID), lambda i, j: (0, 0)),
          pl.BlockSpec((_HID, 128), lambda i, j: (0, 0)),
          pl.BlockSpec((_HID, 128), lambda i, j: (0, 0)),
      ],
      out_specs=[
          pl.BlockSpec((_RB, _HID), lambda i, j: (i, 0)),
          pl.BlockSpec((_RB, 128), lambda i, j: (i, 0)),
          pl.BlockSpec((_RB, 128), lambda i, j: (i, 0)),
      ],
      out_shape=[
          jax.ShapeDtypeStruct((_N, _HID), jnp.float32),
          jax.ShapeDtypeStruct((_N, 128), jnp.float32),
          jax.ShapeDtypeStruct((_N, 128), jnp.float32),
      ],
  )(acc1, den1, W2, b2, A2d, A2s)


def _tc3_body(acc_ref, den_ref, batch_ref, wfc_ref, bfc_ref, out_ref,
              pooled_ref, cnt_ref):
  i = pl.program_id(0)
  j = pl.program_id(1)
  den = den_ref[0] + den_ref[1]
  h = (acc_ref[0, 0] + acc_ref[0, 1]) / (den[:, 0:1] + 1e-16)
  h = jnp.where(h > 0, h, jnp.exp(h) - 1.0)
  b = batch_ref[...].reshape(1, _RB)
  gids = lax.broadcasted_iota(jnp.int32, (_NGRAPH, _RB), 0)
  onehot = jnp.where(gids == b, 1.0, 0.0).astype(jnp.float32)
  v = jnp.dot(onehot, h, preferred_element_type=jnp.float32)

  @pl.when(i == 0)
  def _init():
    pooled_ref[pl.ds(j, 1)] = v.reshape(1, _NGRAPH, 128)

  @pl.when(i > 0)
  def _acc():
    pooled_ref[pl.ds(j, 1)] += v.reshape(1, _NGRAPH, 128)

  @pl.when(j == 0)
  def _cnt():
    c = jnp.broadcast_to(
        jnp.sum(onehot, axis=1, keepdims=True), (_NGRAPH, 128))

    @pl.when(i == 0)
    def _ci():
      cnt_ref[...] = c

    @pl.when(i > 0)
    def _ca():
      cnt_ref[...] += c

  @pl.when((i == _NRB - 1) & (j == 1))
  def _fin():
    pm = jnp.concatenate([pooled_ref[0], pooled_ref[1]], axis=1)
    cnt = jnp.maximum(cnt_ref[...][:, 0:1], 1.0)
    pm = pm / cnt
    out_ref[...] = (
        jnp.dot(pm, wfc_ref[...], preferred_element_type=jnp.float32)
        + bfc_ref[...])


def _tc3(acc2, den2, batch3d, Wfc_p, bfc_p):
  return pl.pallas_call(
      _tc3_body,
      grid=(_NRB, 2),
      in_specs=[
          pl.BlockSpec((1, 2, _RB, 128), lambda i, j: (j, 0, i, 0)),
          pl.BlockSpec((2, _RB, 128), lambda i, j: (0, i, 0)),
          pl.BlockSpec((1, 1, _RB), lambda i, j: (i, 0, 0)),
          pl.BlockSpec((_HID, 128), lambda i, j: (0, 0)),
          pl.BlockSpec((1, 128), lambda i, j: (0, 0)),
      ],
      out_specs=pl.BlockSpec((_NGRAPH, 128), lambda i, j: (0, 0)),
      out_shape=jax.ShapeDtypeStruct((_NGRAPH, 128), jnp.float32),
      scratch_shapes=[
          pltpu.VMEM((2, _NGRAPH, 128), jnp.float32),
          pltpu.VMEM((_NGRAPH, 128), jnp.float32),
      ],
  )(acc2, den2, batch3d, Wfc_p, bfc_p)


_ROWS_PT = _NP // 16
_G = 32
_NGRP = _EPT // _G
_GC = 8
_NCH = _NGRP // _GC


def _make_edge_kernel(layer):
  pps = 16 if layer == 1 else 2
  row_w = 128 if layer == 1 else 256
  halfc = _NCH // 2
  mesh = plsc.VectorSubcoreMesh(core_axis_name="c", subcore_axis_name="s")

  @functools.partial(
      pl.kernel,
      out_type=[
          jax.ShapeDtypeStruct((pps, 2, _NP, 128), jnp.float32),
          jax.ShapeDtypeStruct((2, _NP, 128), jnp.float32),
      ],
      mesh=mesh,
      scratch_types=[
          pltpu.VMEM((_GC, _G), jnp.int32),
          pltpu.VMEM((_GC, _G), jnp.int32),
          pltpu.VMEM((_G, row_w), jnp.float32),
          pltpu.VMEM((_G, 128), jnp.float32),
          pltpu.VMEM((_G, 128), jnp.float32),
          pltpu.VMEM((_G, 128), jnp.float32),
          pltpu.VMEM((8, 128), jnp.float32),
          pltpu.VMEM_SHARED((_NP, 128), jnp.float32),
          pltpu.SemaphoreType.DMA,
      ],
  )
  def edge_kernel(src3_hbm, dst3_hbm, h_hbm, adw_hbm, asw_hbm,
                  acc_hbm, den_hbm,
                  src_v, dst_v, rows_v, prod_v, adw_v, asw_v, zb_v,
                  acc_sh, sem):
    core = lax.axis_index("c")
    sid = lax.axis_index("s")

    def _z(r, _):
      def _zc(c, _2):
        zb_v[r, pl.ds(c * 16, 16)] = jnp.zeros((16,), jnp.float32)
        return 0
      lax.fori_loop(0, 8, _zc, 0)
      return 0
    lax.fori_loop(0, 8, _z, 0)

    def _zero_panel():
      def _za(kk, _):
        pltpu.sync_copy(zb_v, acc_sh.at[pl.ds(sid * _ROWS_PT + kk * 8, 8)])
        return 0
      lax.fori_loop(0, _ROWS_PT // 8, _za, 0)

    def _stage(c2):
      pltpu.sync_copy(src3_hbm.at[sid].at[pl.ds(c2 * _GC, _GC)], src_v)
      pltpu.sync_copy(dst3_hbm.at[sid].at[pl.ds(c2 * _GC, _GC)], dst_v)

    for p in range(pps):
      head = p // 2 if layer == 1 else 0
      col0 = 0 if layer == 1 else (p % 2) * 128

      _zero_panel()
      plsc.subcore_barrier()

      def do_chunk(c2, _):
        _stage(c2)

        def do_group(g, _2):
          sidx = src_v.at[g]
          didx = dst_v.at[g]
          if layer == 1:
            pltpu.async_copy(h_hbm.at[p].at[sidx], rows_v, sem).wait()
          else:
            pltpu.async_copy(h_hbm.at[sidx], rows_v, sem).wait()
          pltpu.async_copy(adw_hbm.at[didx], adw_v, sem).wait()
          pltpu.async_copy(asw_hbm.at[sidx], asw_v, sem).wait()

          def mrow(r, _3):
            e = adw_v[r, pl.ds(0, 16)] + asw_v[r, pl.ds(0, 16)]
            e = jnp.where(e > 0, e, 0.2 * e)
            w16 = jnp.exp(e)
            wsp = jnp.full((16,), w16[head], jnp.float32)
            for c in range(8):
              prod_v[r, pl.ds(c * 16, 16)] = (
                  rows_v[r, pl.ds(col0 + c * 16, 16)] * wsp)
            return 0
          lax.fori_loop(0, _G, mrow, 0)
          pltpu.sync_copy(prod_v, acc_sh.at[didx], add=True)
          return 0
        lax.fori_loop(0, _GC, do_group, 0)
        return 0
      lax.fori_loop(core * halfc, (core + 1) * halfc, do_chunk, 0)

      plsc.subcore_barrier()
      pltpu.sync_copy(
          acc_sh.at[pl.ds(sid * _ROWS_PT, _ROWS_PT)],
          acc_hbm.at[p].at[core].at[pl.ds(sid * _ROWS_PT, _ROWS_PT)])
      plsc.subcore_barrier()

    def _zp(r, _):
      def _zc2(c, _2):
        prod_v[r, pl.ds(c * 16, 16)] = jnp.zeros((16,), jnp.float32)
        return 0
      lax.fori_loop(0, 8, _zc2, 0)
      return 0
    lax.fori_loop(0, _G, _zp, 0)

    _zero_panel()
    plsc.subcore_barrier()

    def den_chunk(c2, _):
      _stage(c2)

      def den_group(g, _2):
        sidx = src_v.at[g]
        didx = dst_v.at[g]
        pltpu.async_copy(adw_hbm.at[didx], adw_v, sem).wait()
        pltpu.async_copy(asw_hbm.at[sidx], asw_v, sem).wait()

        def wrow(r, _3):
          e = adw_v[r, pl.ds(0, 16)] + asw_v[r, pl.ds(0, 16)]
          e = jnp.where(e > 0, e, 0.2 * e)
          prod_v[r, pl.ds(0, 16)] = jnp.exp(e)
          return 0
        lax.fori_loop(0, _G, wrow, 0)
        pltpu.sync_copy(prod_v, acc_sh.at[didx], add=True)
        return 0
      lax.fori_loop(0, _GC, den_group, 0)
      return 0
    lax.fori_loop(core * halfc, (core + 1) * halfc, den_chunk, 0)

    plsc.subcore_barrier()
    pltpu.sync_copy(
        acc_sh.at[pl.ds(sid * _ROWS_PT, _ROWS_PT)],
        den_hbm.at[core].at[pl.ds(sid * _ROWS_PT, _ROWS_PT)])

  return edge_kernel


_edge_l1 = _make_edge_kernel(1)
_edge_l2 = _make_edge_kernel(2)


def _pad_rows(a, rows):
  return jnp.concatenate(
      [a, jnp.zeros((rows - a.shape[0],) + a.shape[1:], a.dtype)], axis=0)


def _score_tables(ad, asr):
  return _pad_rows(ad, _NT), _pad_rows(asr, _NT)


@jax.jit
def kernel(x, edge_index, batch, W1, b1, att1, W2, b2, att2, Wfc, bfc):
  loops = jnp.arange(_N, dtype=edge_index.dtype)
  src = jnp.concatenate(
      [edge_index[0], loops,
       jnp.zeros((_E_PAD - _E_FULL,), edge_index.dtype)])
  dst = jnp.concatenate(
      [edge_index[1], loops,
       jnp.full((_E_PAD - _E_FULL,), _N, edge_index.dtype)])
  src3 = src.reshape(16, _NGRP, _G)
  dst3 = dst.reshape(16, _NGRP, _G)

  def att_mats(att, heads):
    ad = att[:, :_HID]
    asr = att[:, _HID:]

    def expand(a):
      m = jnp.zeros((heads * _HID, 128), jnp.float32)
      for h in range(heads):
        m = m.at[h * _HID:(h + 1) * _HID, h].set(a[h])
      return m
    return expand(ad), expand(asr)

  A1d, A1s = att_mats(att1, _HEADS)
  A2d, A2s = att_mats(att2, 1)

  hp1, ad1, as1 = _tc1(x, W1, b1.reshape(1, -1), A1d, A1s)
  adw1, asw1 = _score_tables(ad1, as1)

  acc1, den1 = _edge_l1(src3, dst3, hp1, adw1, asw1)

  h2, ad2, as2 = _tc2(acc1, den1, W2, b2.reshape(1, -1), A2d, A2s)
  adw2, asw2 = _score_tables(ad2, as2)

  acc2, den2 = _edge_l2(src3, dst3, h2, adw2, asw2)

  batch3d = batch.reshape(_NRB, 1, _RB)
  Wfc_p = jnp.concatenate(
      [Wfc, jnp.zeros((_HID, 127), jnp.float32)], axis=1)
  bfc_p = jnp.concatenate(
      [bfc.reshape(1, 1), jnp.zeros((1, 127), jnp.float32)], axis=1)
  res = _tc3(acc2, den2, batch3d, Wfc_p, bfc_p)
  return res[:, :1]


# speedup vs baseline: 3.5796x; 1.7600x over previous
import functools

import jax
import jax.numpy as jnp
from jax import lax
from jax.experimental import pallas as pl
from jax.experimental.pallas import tpu as pltpu
from jax.experimental.pallas import tpu_sc as plsc

_N = 10000
_D = 256
_HID = 256
_HEADS = 8
_NGRAPH = 8

_E_RAW = 160000
_E_FULL = _E_RAW + _N
_E_PAD = 172032
_EPT = _E_PAD // 16
_G = 64
_NGRP = _EPT // _G

_NP = 10240
_NT = 10016

_RB = 400
_NRB = _N // _RB


def _tc1_body(x_ref, w_ref, b_ref, a1d_ref, a1s_ref, hp_ref, ad_ref, as_ref):
  j = pl.program_id(1)
  hp = jnp.dot(x_ref[...], w_ref[...], preferred_element_type=jnp.float32)
  hp = hp + b_ref[...]
  hp_ref[...] = hp.reshape(1, _RB, 128)
  ad = jnp.dot(hp, a1d_ref[...], preferred_element_type=jnp.float32)
  asr = jnp.dot(hp, a1s_ref[...], preferred_element_type=jnp.float32)

  @pl.when(j == 0)
  def _init():
    ad_ref[...] = ad
    as_ref[...] = asr

  @pl.when(j > 0)
  def _acc():
    ad_ref[...] += ad
    as_ref[...] += asr


def _tc1(x, W1, b1, A1d, A1s):
  return pl.pallas_call(
      _tc1_body,
      grid=(_NRB, 16),
      in_specs=[
          pl.BlockSpec((_RB, _D), lambda i, j: (i, 0)),
          pl.BlockSpec((_D, 128), lambda i, j: (0, j)),
          pl.BlockSpec((1, 128), lambda i, j: (0, j)),
          pl.BlockSpec((128, 128), lambda i, j: (j, 0)),
          pl.BlockSpec((128, 128), lambda i, j: (j, 0)),
      ],
      out_specs=[
          pl.BlockSpec((1, _RB, 128), lambda i, j: (j, i, 0)),
          pl.BlockSpec((_RB, 128), lambda i, j: (i, 0)),
          pl.BlockSpec((_RB, 128), lambda i, j: (i, 0)),
      ],
      out_shape=[
          jax.ShapeDtypeStruct((16, _N, 128), jnp.float32),
          jax.ShapeDtypeStruct((_N, 128), jnp.float32),
          jax.ShapeDtypeStruct((_N, 128), jnp.float32),
      ],
  )(x, W1, b1, A1d, A1s)


def _tc2_body(acc_ref, den_ref, w_ref, b_ref, a2d_ref, a2s_ref,
              h2_ref, ad_ref, as_ref):
  j = pl.program_id(1)
  den = den_ref[0] + den_ref[1]
  cmask = lax.broadcasted_iota(jnp.int32, (_RB, 128), 1) == (j // 2)
  dcol = jnp.sum(jnp.where(cmask, den, 0.0), axis=1, keepdims=True)
  hin = (acc_ref[0, 0] + acc_ref[0, 1]) / (dcol + 1e-16)
  hin = jnp.where(hin > 0, hin, jnp.exp(hin) - 1.0)
  part = jnp.dot(hin, w_ref[...], preferred_element_type=jnp.float32)

  @pl.when(j == 0)
  def _init():
    h2_ref[...] = part + b_ref[...]

  @pl.when(j > 0)
  def _acc():
    h2_ref[...] += part

  @pl.when(j == 15)
  def _fin():
    h2 = h2_ref[...]
    ad_ref[...] = jnp.dot(h2, a2d_ref[...], preferred_element_type=jnp.float32)
    as_ref[...] = jnp.dot(h2, a2s_ref[...], preferred_element_type=jnp.float32)


def _tc2(acc1, den1, W2, b2, A2d, A2s):
  return pl.pallas_call(
      _tc2_body,
      grid=(_NRB, 16),
      in_specs=[
          pl.BlockSpec((1, 2, _RB, 128), lambda i, j: (j, 0, i, 0)),
          pl.BlockSpec((2, _RB, 128), lambda i, j: (0, i, 0)),
          pl.BlockSpec((128, _HID), lambda i, j: (j, 0)),
          pl.BlockSpec((1, _HID), lambda i, j: (0, 0)),
          pl.BlockSpec((_HID, 128), lambda i, j: (0, 0)),
          pl.BlockSpec((_HID, 128), lambda i, j: (0, 0)),
      ],
      out_specs=[
          pl.BlockSpec((_RB, _HID), lambda i, j: (i, 0)),
          pl.BlockSpec((_RB, 128), lambda i, j: (i, 0)),
          pl.BlockSpec((_RB, 128), lambda i, j: (i, 0)),
      ],
      out_shape=[
          jax.ShapeDtypeStruct((_N, _HID), jnp.float32),
          jax.ShapeDtypeStruct((_N, 128), jnp.float32),
          jax.ShapeDtypeStruct((_N, 128), jnp.float32),
      ],
  )(acc1, den1, W2, b2, A2d, A2s)


def _tc3_body(acc_ref, den_ref, batch_ref, wfc_ref, bfc_ref, out_ref,
              pooled_ref, cnt_ref):
  i = pl.program_id(0)
  j = pl.program_id(1)
  den = den_ref[0] + den_ref[1]
  h = (acc_ref[0, 0] + acc_ref[0, 1]) / (den[:, 0:1] + 1e-16)
  h = jnp.where(h > 0, h, jnp.exp(h) - 1.0)
  b = batch_ref[...].reshape(1, _RB)
  gids = lax.broadcasted_iota(jnp.int32, (_NGRAPH, _RB), 0)
  onehot = jnp.where(gids == b, 1.0, 0.0).astype(jnp.float32)
  v = jnp.dot(onehot, h, preferred_element_type=jnp.float32)

  @pl.when(i == 0)
  def _init():
    pooled_ref[pl.ds(j, 1)] = v.reshape(1, _NGRAPH, 128)

  @pl.when(i > 0)
  def _acc():
    pooled_ref[pl.ds(j, 1)] += v.reshape(1, _NGRAPH, 128)

  @pl.when(j == 0)
  def _cnt():
    c = jnp.broadcast_to(
        jnp.sum(onehot, axis=1, keepdims=True), (_NGRAPH, 128))

    @pl.when(i == 0)
    def _ci():
      cnt_ref[...] = c

    @pl.when(i > 0)
    def _ca():
      cnt_ref[...] += c

  @pl.when((i == _NRB - 1) & (j == 1))
  def _fin():
    pm = jnp.concatenate([pooled_ref[0], pooled_ref[1]], axis=1)
    cnt = jnp.maximum(cnt_ref[...][:, 0:1], 1.0)
    pm = pm / cnt
    out_ref[...] = (
        jnp.dot(pm, wfc_ref[...], preferred_element_type=jnp.float32)
        + bfc_ref[...])


def _tc3(acc2, den2, batch3d, Wfc_p, bfc_p):
  return pl.pallas_call(
      _tc3_body,
      grid=(_NRB, 2),
      in_specs=[
          pl.BlockSpec((1, 2, _RB, 128), lambda i, j: (j, 0, i, 0)),
          pl.BlockSpec((2, _RB, 128), lambda i, j: (0, i, 0)),
          pl.BlockSpec((1, 1, _RB), lambda i, j: (i, 0, 0)),
          pl.BlockSpec((_HID, 128), lambda i, j: (0, 0)),
          pl.BlockSpec((1, 128), lambda i, j: (0, 0)),
      ],
      out_specs=pl.BlockSpec((_NGRAPH, 128), lambda i, j: (0, 0)),
      out_shape=jax.ShapeDtypeStruct((_NGRAPH, 128), jnp.float32),
      scratch_shapes=[
          pltpu.VMEM((2, _NGRAPH, 128), jnp.float32),
          pltpu.VMEM((_NGRAPH, 128), jnp.float32),
      ],
  )(acc2, den2, batch3d, Wfc_p, bfc_p)


_ROWS_PT = _NP // 16
_G = 32
_NGRP = _EPT // _G
_GC = 8
_NCH = _NGRP // _GC


def _make_edge_kernel(layer):
  pps = 16 if layer == 1 else 2
  row_w = 128 if layer == 1 else 256
  halfc = _NCH // 2
  mesh = plsc.VectorSubcoreMesh(core_axis_name="c", subcore_axis_name="s")

  @functools.partial(
      pl.kernel,
      out_type=[
          jax.ShapeDtypeStruct((pps, 2, _NP, 128), jnp.float32),
          jax.ShapeDtypeStruct((2, _NP, 128), jnp.float32),
      ],
      mesh=mesh,
      scratch_types=[
          pltpu.VMEM((_GC, _G), jnp.int32),
          pltpu.VMEM((_GC, _G), jnp.int32),
          pltpu.VMEM((_G, row_w), jnp.float32),
          pltpu.VMEM((_G, 128), jnp.float32),
          pltpu.VMEM((_G, 128), jnp.float32),
          pltpu.VMEM((_G, 128), jnp.float32),
          pltpu.VMEM((8, 128), jnp.float32),
          pltpu.VMEM_SHARED((_NP, 128), jnp.float32),
          pltpu.SemaphoreType.DMA,
      ],
  )
  def edge_kernel(src3_hbm, dst3_hbm, h_hbm, adw_hbm, asw_hbm,
                  acc_hbm, den_hbm,
                  src_v, dst_v, rows_v, prod_v, adw_v, asw_v, zb_v,
                  acc_sh, sem):
    core = lax.axis_index("c")
    sid = lax.axis_index("s")

    def _z(r, _):
      def _zc(c, _2):
        zb_v[r, pl.ds(c * 16, 16)] = jnp.zeros((16,), jnp.float32)
        return 0
      lax.fori_loop(0, 8, _zc, 0)
      return 0
    lax.fori_loop(0, 8, _z, 0)

    def _zero_panel():
      def _za(kk, _):
        pltpu.sync_copy(zb_v, acc_sh.at[pl.ds(sid * _ROWS_PT + kk * 8, 8)])
        return 0
      lax.fori_loop(0, _ROWS_PT // 8, _za, 0)

    def _stage(c2):
      pltpu.sync_copy(src3_hbm.at[sid].at[pl.ds(c2 * _GC, _GC)], src_v)
      pltpu.sync_copy(dst3_hbm.at[sid].at[pl.ds(c2 * _GC, _GC)], dst_v)

    for p in range(pps):
      head = p // 2 if layer == 1 else 0
      col0 = 0 if layer == 1 else (p % 2) * 128

      _zero_panel()
      plsc.subcore_barrier()

      def do_chunk(c2, _):
        _stage(c2)

        def do_group(g, _2):
          sidx = src_v.at[g]
          didx = dst_v.at[g]
          if layer == 1:
            c1 = pltpu.async_copy(h_hbm.at[p].at[sidx], rows_v, sem)
          else:
            c1 = pltpu.async_copy(h_hbm.at[sidx], rows_v, sem)
          c2 = pltpu.async_copy(adw_hbm.at[didx], adw_v, sem)
          c3 = pltpu.async_copy(asw_hbm.at[sidx], asw_v, sem)
          c1.wait()
          c2.wait()
          c3.wait()

          def mrow(r, _3):
            e = adw_v[r, pl.ds(0, 16)] + asw_v[r, pl.ds(0, 16)]
            e = jnp.where(e > 0, e, 0.2 * e)
            w16 = jnp.exp(e)
            wsp = jnp.full((16,), w16[head], jnp.float32)
            for c in range(8):
              prod_v[r, pl.ds(c * 16, 16)] = (
                  rows_v[r, pl.ds(col0 + c * 16, 16)] * wsp)
            return 0
          lax.fori_loop(0, _G, mrow, 0)
          pltpu.sync_copy(prod_v, acc_sh.at[didx], add=True)
          return 0
        lax.fori_loop(0, _GC, do_group, 0)
        return 0
      lax.fori_loop(core * halfc, (core + 1) * halfc, do_chunk, 0)

      plsc.subcore_barrier()
      pltpu.sync_copy(
          acc_sh.at[pl.ds(sid * _ROWS_PT, _ROWS_PT)],
          acc_hbm.at[p].at[core].at[pl.ds(sid * _ROWS_PT, _ROWS_PT)])
      plsc.subcore_barrier()

    def _zp(r, _):
      def _zc2(c, _2):
        prod_v[r, pl.ds(c * 16, 16)] = jnp.zeros((16,), jnp.float32)
        return 0
      lax.fori_loop(0, 8, _zc2, 0)
      return 0
    lax.fori_loop(0, _G, _zp, 0)

    _zero_panel()
    plsc.subcore_barrier()

    def den_chunk(c2, _):
      _stage(c2)

      def den_group(g, _2):
        sidx = src_v.at[g]
        didx = dst_v.at[g]
        c2 = pltpu.async_copy(adw_hbm.at[didx], adw_v, sem)
        c3 = pltpu.async_copy(asw_hbm.at[sidx], asw_v, sem)
        c2.wait()
        c3.wait()

        def wrow(r, _3):
          e = adw_v[r, pl.ds(0, 16)] + asw_v[r, pl.ds(0, 16)]
          e = jnp.where(e > 0, e, 0.2 * e)
          prod_v[r, pl.ds(0, 16)] = jnp.exp(e)
          return 0
        lax.fori_loop(0, _G, wrow, 0)
        pltpu.sync_copy(prod_v, acc_sh.at[didx], add=True)
        return 0
      lax.fori_loop(0, _GC, den_group, 0)
      return 0
    lax.fori_loop(core * halfc, (core + 1) * halfc, den_chunk, 0)

    plsc.subcore_barrier()
    pltpu.sync_copy(
        acc_sh.at[pl.ds(sid * _ROWS_PT, _ROWS_PT)],
        den_hbm.at[core].at[pl.ds(sid * _ROWS_PT, _ROWS_PT)])

  return edge_kernel


_edge_l1 = _make_edge_kernel(1)
_edge_l2 = _make_edge_kernel(2)


def _pad_rows(a, rows):
  return jnp.concatenate(
      [a, jnp.zeros((rows - a.shape[0],) + a.shape[1:], a.dtype)], axis=0)


def _score_tables(ad, asr):
  return _pad_rows(ad, _NT), _pad_rows(asr, _NT)


@jax.jit
def kernel(x, edge_index, batch, W1, b1, att1, W2, b2, att2, Wfc, bfc):
  loops = jnp.arange(_N, dtype=edge_index.dtype)
  src = jnp.concatenate(
      [edge_index[0], loops,
       jnp.zeros((_E_PAD - _E_FULL,), edge_index.dtype)])
  dst = jnp.concatenate(
      [edge_index[1], loops,
       jnp.full((_E_PAD - _E_FULL,), _N, edge_index.dtype)])
  src3 = src.reshape(16, _NGRP, _G)
  dst3 = dst.reshape(16, _NGRP, _G)

  def att_mats(att, heads):
    ad = att[:, :_HID]
    asr = att[:, _HID:]

    def expand(a):
      m = jnp.zeros((heads * _HID, 128), jnp.float32)
      for h in range(heads):
        m = m.at[h * _HID:(h + 1) * _HID, h].set(a[h])
      return m
    return expand(ad), expand(asr)

  A1d, A1s = att_mats(att1, _HEADS)
  A2d, A2s = att_mats(att2, 1)

  hp1, ad1, as1 = _tc1(x, W1, b1.reshape(1, -1), A1d, A1s)
  adw1, asw1 = _score_tables(ad1, as1)

  acc1, den1 = _edge_l1(src3, dst3, hp1, adw1, asw1)

  h2, ad2, as2 = _tc2(acc1, den1, W2, b2.reshape(1, -1), A2d, A2s)
  adw2, asw2 = _score_tables(ad2, as2)

  acc2, den2 = _edge_l2(src3, dst3, h2, adw2, asw2)

  batch3d = batch.reshape(_NRB, 1, _RB)
  Wfc_p = jnp.concatenate(
      [Wfc, jnp.zeros((_HID, 127), jnp.float32)], axis=1)
  bfc_p = jnp.concatenate(
      [bfc.reshape(1, 1), jnp.zeros((1, 127), jnp.float32)], axis=1)
  res = _tc3(acc2, den2, batch3d, Wfc_p, bfc_p)
  return res[:, :1]


# speedup vs baseline: 3.9478x; 1.1029x over previous
import functools

import jax
import jax.numpy as jnp
from jax import lax
from jax.experimental import pallas as pl
from jax.experimental.pallas import tpu as pltpu
from jax.experimental.pallas import tpu_sc as plsc

_N = 10000
_D = 256
_HID = 256
_HEADS = 8
_NGRAPH = 8

_E_RAW = 160000
_E_FULL = _E_RAW + _N
_E_PAD = 172032
_EPT = _E_PAD // 16
_G = 64
_NGRP = _EPT // _G

_NP = 10240
_NT = 10016

_RB = 400
_NRB = _N // _RB


def _tc1_body(x_ref, w_ref, b_ref, a1d_ref, a1s_ref, hp_ref, ad_ref, as_ref):
  j = pl.program_id(1)
  hp = jnp.dot(x_ref[...], w_ref[...], preferred_element_type=jnp.float32)
  hp = hp + b_ref[...]
  hp_ref[...] = hp.reshape(1, _RB, 128)
  ad = jnp.dot(hp, a1d_ref[...], preferred_element_type=jnp.float32)
  asr = jnp.dot(hp, a1s_ref[...], preferred_element_type=jnp.float32)

  @pl.when(j == 0)
  def _init():
    ad_ref[...] = ad
    as_ref[...] = asr

  @pl.when(j > 0)
  def _acc():
    ad_ref[...] += ad
    as_ref[...] += asr


def _tc1(x, W1, b1, A1d, A1s):
  return pl.pallas_call(
      _tc1_body,
      grid=(_NRB, 16),
      in_specs=[
          pl.BlockSpec((_RB, _D), lambda i, j: (i, 0)),
          pl.BlockSpec((_D, 128), lambda i, j: (0, j)),
          pl.BlockSpec((1, 128), lambda i, j: (0, j)),
          pl.BlockSpec((128, 128), lambda i, j: (j, 0)),
          pl.BlockSpec((128, 128), lambda i, j: (j, 0)),
      ],
      out_specs=[
          pl.BlockSpec((1, _RB, 128), lambda i, j: (j, i, 0)),
          pl.BlockSpec((_RB, 128), lambda i, j: (i, 0)),
          pl.BlockSpec((_RB, 128), lambda i, j: (i, 0)),
      ],
      out_shape=[
          jax.ShapeDtypeStruct((16, _N, 128), jnp.float32),
          jax.ShapeDtypeStruct((_N, 128), jnp.float32),
          jax.ShapeDtypeStruct((_N, 128), jnp.float32),
      ],
  )(x, W1, b1, A1d, A1s)


def _tc2_body(acc_ref, den_ref, w_ref, b_ref, a2d_ref, a2s_ref,
              h2_ref, ad_ref, as_ref):
  j = pl.program_id(1)
  den = den_ref[0] + den_ref[1]
  cmask = lax.broadcasted_iota(jnp.int32, (_RB, 128), 1) == (j // 2)
  dcol = jnp.sum(jnp.where(cmask, den, 0.0), axis=1, keepdims=True)
  hin = (acc_ref[0, 0] + acc_ref[0, 1]) / (dcol + 1e-16)
  hin = jnp.where(hin > 0, hin, jnp.exp(hin) - 1.0)
  part = jnp.dot(hin, w_ref[...], preferred_element_type=jnp.float32)

  @pl.when(j == 0)
  def _init():
    h2_ref[...] = part + b_ref[...]

  @pl.when(j > 0)
  def _acc():
    h2_ref[...] += part

  @pl.when(j == 15)
  def _fin():
    h2 = h2_ref[...]
    ad_ref[...] = jnp.dot(h2, a2d_ref[...], preferred_element_type=jnp.float32)
    as_ref[...] = jnp.dot(h2, a2s_ref[...], preferred_element_type=jnp.float32)


def _tc2(acc1, den1, W2, b2, A2d, A2s):
  return pl.pallas_call(
      _tc2_body,
      grid=(_NRB, 16),
      in_specs=[
          pl.BlockSpec((1, 2, _RB, 128), lambda i, j: (j, 0, i, 0)),
          pl.BlockSpec((2, _RB, 128), lambda i, j: (0, i, 0)),
          pl.BlockSpec((128, _HID), lambda i, j: (j, 0)),
          pl.BlockSpec((1, _HID), lambda i, j: (0, 0)),
          pl.BlockSpec((_HID, 128), lambda i, j: (0, 0)),
          pl.BlockSpec((_HID, 128), lambda i, j: (0, 0)),
      ],
      out_specs=[
          pl.BlockSpec((_RB, _HID), lambda i, j: (i, 0)),
          pl.BlockSpec((_RB, 128), lambda i, j: (i, 0)),
          pl.BlockSpec((_RB, 128), lambda i, j: (i, 0)),
      ],
      out_shape=[
          jax.ShapeDtypeStruct((_N, _HID), jnp.float32),
          jax.ShapeDtypeStruct((_N, 128), jnp.float32),
          jax.ShapeDtypeStruct((_N, 128), jnp.float32),
      ],
  )(acc1, den1, W2, b2, A2d, A2s)


def _tc3_body(acc_ref, den_ref, batch_ref, wfc_ref, bfc_ref, out_ref,
              pooled_ref, cnt_ref):
  i = pl.program_id(0)
  j = pl.program_id(1)
  den = den_ref[0] + den_ref[1]
  h = (acc_ref[0, 0] + acc_ref[0, 1]) / (den[:, 0:1] + 1e-16)
  h = jnp.where(h > 0, h, jnp.exp(h) - 1.0)
  b = batch_ref[...].reshape(1, _RB)
  gids = lax.broadcasted_iota(jnp.int32, (_NGRAPH, _RB), 0)
  onehot = jnp.where(gids == b, 1.0, 0.0).astype(jnp.float32)
  v = jnp.dot(onehot, h, preferred_element_type=jnp.float32)

  @pl.when(i == 0)
  def _init():
    pooled_ref[pl.ds(j, 1)] = v.reshape(1, _NGRAPH, 128)

  @pl.when(i > 0)
  def _acc():
    pooled_ref[pl.ds(j, 1)] += v.reshape(1, _NGRAPH, 128)

  @pl.when(j == 0)
  def _cnt():
    c = jnp.broadcast_to(
        jnp.sum(onehot, axis=1, keepdims=True), (_NGRAPH, 128))

    @pl.when(i == 0)
    def _ci():
      cnt_ref[...] = c

    @pl.when(i > 0)
    def _ca():
      cnt_ref[...] += c

  @pl.when((i == _NRB - 1) & (j == 1))
  def _fin():
    pm = jnp.concatenate([pooled_ref[0], pooled_ref[1]], axis=1)
    cnt = jnp.maximum(cnt_ref[...][:, 0:1], 1.0)
    pm = pm / cnt
    out_ref[...] = (
        jnp.dot(pm, wfc_ref[...], preferred_element_type=jnp.float32)
        + bfc_ref[...])


def _tc3(acc2, den2, batch3d, Wfc_p, bfc_p):
  return pl.pallas_call(
      _tc3_body,
      grid=(_NRB, 2),
      in_specs=[
          pl.BlockSpec((1, 2, _RB, 128), lambda i, j: (j, 0, i, 0)),
          pl.BlockSpec((2, _RB, 128), lambda i, j: (0, i, 0)),
          pl.BlockSpec((1, 1, _RB), lambda i, j: (i, 0, 0)),
          pl.BlockSpec((_HID, 128), lambda i, j: (0, 0)),
          pl.BlockSpec((1, 128), lambda i, j: (0, 0)),
      ],
      out_specs=pl.BlockSpec((_NGRAPH, 128), lambda i, j: (0, 0)),
      out_shape=jax.ShapeDtypeStruct((_NGRAPH, 128), jnp.float32),
      scratch_shapes=[
          pltpu.VMEM((2, _NGRAPH, 128), jnp.float32),
          pltpu.VMEM((_NGRAPH, 128), jnp.float32),
      ],
  )(acc2, den2, batch3d, Wfc_p, bfc_p)


_ROWS_PT = _NP // 16
_G = 48
_NGRP = _EPT // _G
_GC = 8
_NCH = _NGRP // _GC


def _make_edge_kernel(layer):
  pps = 16 if layer == 1 else 2
  row_w = 128 if layer == 1 else 256
  halfc = _NCH // 2
  mesh = plsc.VectorSubcoreMesh(core_axis_name="c", subcore_axis_name="s")

  @functools.partial(
      pl.kernel,
      out_type=[
          jax.ShapeDtypeStruct((pps, 2, _NP, 128), jnp.float32),
          jax.ShapeDtypeStruct((2, _NP, 128), jnp.float32),
      ],
      mesh=mesh,
      scratch_types=[
          pltpu.VMEM((_GC, _G), jnp.int32),
          pltpu.VMEM((_GC, _G), jnp.int32),
          pltpu.VMEM((_G, row_w), jnp.float32),
          pltpu.VMEM((_G, 128), jnp.float32),
          pltpu.VMEM((_G, 128), jnp.float32),
          pltpu.VMEM((_G, 128), jnp.float32),
          pltpu.VMEM((8, 128), jnp.float32),
          pltpu.VMEM_SHARED((_NP, 128), jnp.float32),
          pltpu.SemaphoreType.DMA,
      ],
  )
  def edge_kernel(src3_hbm, dst3_hbm, h_hbm, adw_hbm, asw_hbm,
                  acc_hbm, den_hbm,
                  src_v, dst_v, rows_v, prod_v, adw_v, asw_v, zb_v,
                  acc_sh, sem):
    core = lax.axis_index("c")
    sid = lax.axis_index("s")

    def _z(r, _):
      def _zc(c, _2):
        zb_v[r, pl.ds(c * 16, 16)] = jnp.zeros((16,), jnp.float32)
        return 0
      lax.fori_loop(0, 8, _zc, 0)
      return 0
    lax.fori_loop(0, 8, _z, 0)

    def _zero_panel():
      def _za(kk, _):
        pltpu.sync_copy(zb_v, acc_sh.at[pl.ds(sid * _ROWS_PT + kk * 8, 8)])
        return 0
      lax.fori_loop(0, _ROWS_PT // 8, _za, 0)

    def _stage(c2):
      pltpu.sync_copy(src3_hbm.at[sid].at[pl.ds(c2 * _GC, _GC)], src_v)
      pltpu.sync_copy(dst3_hbm.at[sid].at[pl.ds(c2 * _GC, _GC)], dst_v)

    for p in range(pps):
      head = p // 2 if layer == 1 else 0
      col0 = 0 if layer == 1 else (p % 2) * 128

      _zero_panel()
      plsc.subcore_barrier()

      def do_chunk(c2, _):
        _stage(c2)

        def do_group(g, _2):
          sidx = src_v.at[g]
          didx = dst_v.at[g]
          if layer == 1:
            c1 = pltpu.async_copy(h_hbm.at[p].at[sidx], rows_v, sem)
          else:
            c1 = pltpu.async_copy(h_hbm.at[sidx], rows_v, sem)
          c2a = pltpu.async_copy(adw_hbm.at[didx], adw_v, sem)
          c2b = pltpu.async_copy(asw_hbm.at[sidx], asw_v, sem)
          c1.wait()
          c2a.wait()
          c2b.wait()

          def mrow(r, _3):
            e = adw_v[r, pl.ds(0, 16)] + asw_v[r, pl.ds(0, 16)]
            e = jnp.where(e > 0, e, 0.2 * e)
            w16 = jnp.exp(e)
            wsp = jnp.full((16,), w16[head], jnp.float32)
            for c in range(8):
              prod_v[r, pl.ds(c * 16, 16)] = (
                  rows_v[r, pl.ds(col0 + c * 16, 16)] * wsp)
            return 0
          lax.fori_loop(0, _G, mrow, 0)
          pltpu.sync_copy(prod_v, acc_sh.at[didx], add=True)
          return 0
        lax.fori_loop(0, _GC, do_group, 0)
        return 0
      lax.fori_loop(core * halfc, (core + 1) * halfc, do_chunk, 0)

      plsc.subcore_barrier()
      pltpu.sync_copy(
          acc_sh.at[pl.ds(sid * _ROWS_PT, _ROWS_PT)],
          acc_hbm.at[p].at[core].at[pl.ds(sid * _ROWS_PT, _ROWS_PT)])
      plsc.subcore_barrier()

    def _zp(r, _):
      def _zc2(c, _2):
        prod_v[r, pl.ds(c * 16, 16)] = jnp.zeros((16,), jnp.float32)
        return 0
      lax.fori_loop(0, 8, _zc2, 0)
      return 0
    lax.fori_loop(0, _G, _zp, 0)

    _zero_panel()
    plsc.subcore_barrier()

    def den_chunk(c2, _):
      _stage(c2)

      def den_group(g, _2):
        sidx = src_v.at[g]
        didx = dst_v.at[g]
        c2a = pltpu.async_copy(adw_hbm.at[didx], adw_v, sem)
        c2b = pltpu.async_copy(asw_hbm.at[sidx], asw_v, sem)
        c2a.wait()
        c2b.wait()

        def wrow(r, _3):
          e = adw_v[r, pl.ds(0, 16)] + asw_v[r, pl.ds(0, 16)]
          e = jnp.where(e > 0, e, 0.2 * e)
          prod_v[r, pl.ds(0, 16)] = jnp.exp(e)
          return 0
        lax.fori_loop(0, _G, wrow, 0)
        pltpu.sync_copy(prod_v, acc_sh.at[didx], add=True)
        return 0
      lax.fori_loop(0, _GC, den_group, 0)
      return 0
    lax.fori_loop(core * halfc, (core + 1) * halfc, den_chunk, 0)

    plsc.subcore_barrier()
    pltpu.sync_copy(
        acc_sh.at[pl.ds(sid * _ROWS_PT, _ROWS_PT)],
        den_hbm.at[core].at[pl.ds(sid * _ROWS_PT, _ROWS_PT)])

  return edge_kernel


_edge_l1 = _make_edge_kernel(1)
_edge_l2 = _make_edge_kernel(2)


def _pad_rows(a, rows):
  return jnp.concatenate(
      [a, jnp.zeros((rows - a.shape[0],) + a.shape[1:], a.dtype)], axis=0)


def _score_tables(ad, asr):
  return _pad_rows(ad, _NT), _pad_rows(asr, _NT)


@jax.jit
def kernel(x, edge_index, batch, W1, b1, att1, W2, b2, att2, Wfc, bfc):
  loops = jnp.arange(_N, dtype=edge_index.dtype)
  src = jnp.concatenate(
      [edge_index[0], loops,
       jnp.zeros((_E_PAD - _E_FULL,), edge_index.dtype)])
  dst = jnp.concatenate(
      [edge_index[1], loops,
       jnp.full((_E_PAD - _E_FULL,), _N, edge_index.dtype)])
  src3 = src.reshape(16, _NGRP, _G)
  dst3 = dst.reshape(16, _NGRP, _G)

  def att_mats(att, heads):
    ad = att[:, :_HID]
    asr = att[:, _HID:]

    def expand(a):
      m = jnp.zeros((heads * _HID, 128), jnp.float32)
      for h in range(heads):
        m = m.at[h * _HID:(h + 1) * _HID, h].set(a[h])
      return m
    return expand(ad), expand(asr)

  A1d, A1s = att_mats(att1, _HEADS)
  A2d, A2s = att_mats(att2, 1)

  hp1, ad1, as1 = _tc1(x, W1, b1.reshape(1, -1), A1d, A1s)
  adw1, asw1 = _score_tables(ad1, as1)

  acc1, den1 = _edge_l1(src3, dst3, hp1, adw1, asw1)

  h2, ad2, as2 = _tc2(acc1, den1, W2, b2.reshape(1, -1), A2d, A2s)
  adw2, asw2 = _score_tables(ad2, as2)

  acc2, den2 = _edge_l2(src3, dst3, h2, adw2, asw2)

  batch3d = batch.reshape(_NRB, 1, _RB)
  Wfc_p = jnp.concatenate(
      [Wfc, jnp.zeros((_HID, 127), jnp.float32)], axis=1)
  bfc_p = jnp.concatenate(
      [bfc.reshape(1, 1), jnp.zeros((1, 127), jnp.float32)], axis=1)
  res = _tc3(acc2, den2, batch3d, Wfc_p, bfc_p)
  return res[:, :1]


# speedup vs baseline: 4.0617x; 1.0289x over previous
import functools

import jax
import jax.numpy as jnp
from jax import lax
from jax.experimental import pallas as pl
from jax.experimental.pallas import tpu as pltpu
from jax.experimental.pallas import tpu_sc as plsc

_N = 10000
_D = 256
_HID = 256
_HEADS = 8
_NGRAPH = 8

_E_RAW = 160000
_E_FULL = _E_RAW + _N
_E_PAD = 172032
_EPT = _E_PAD // 16
_G = 64
_NGRP = _EPT // _G

_NP = 10240
_NT = 10016

_RB = 400
_NRB = _N // _RB


def _tc1_body(x_ref, w_ref, b_ref, a1d_ref, a1s_ref, hp_ref, ad_ref, as_ref):
  j = pl.program_id(1)
  hp = jnp.dot(x_ref[...], w_ref[...], preferred_element_type=jnp.float32)
  hp = hp + b_ref[...]
  hp_ref[...] = hp.reshape(1, _RB, 128)
  ad = jnp.dot(hp, a1d_ref[...], preferred_element_type=jnp.float32)
  asr = jnp.dot(hp, a1s_ref[...], preferred_element_type=jnp.float32)

  @pl.when(j == 0)
  def _init():
    ad_ref[...] = ad
    as_ref[...] = asr

  @pl.when(j > 0)
  def _acc():
    ad_ref[...] += ad
    as_ref[...] += asr


def _tc1(x, W1, b1, A1d, A1s):
  return pl.pallas_call(
      _tc1_body,
      grid=(_NRB, 16),
      in_specs=[
          pl.BlockSpec((_RB, _D), lambda i, j: (i, 0)),
          pl.BlockSpec((_D, 128), lambda i, j: (0, j)),
          pl.BlockSpec((1, 128), lambda i, j: (0, j)),
          pl.BlockSpec((128, 128), lambda i, j: (j, 0)),
          pl.BlockSpec((128, 128), lambda i, j: (j, 0)),
      ],
      out_specs=[
          pl.BlockSpec((1, _RB, 128), lambda i, j: (j, i, 0)),
          pl.BlockSpec((_RB, 128), lambda i, j: (i, 0)),
          pl.BlockSpec((_RB, 128), lambda i, j: (i, 0)),
      ],
      out_shape=[
          jax.ShapeDtypeStruct((16, _N, 128), jnp.float32),
          jax.ShapeDtypeStruct((_N, 128), jnp.float32),
          jax.ShapeDtypeStruct((_N, 128), jnp.float32),
      ],
  )(x, W1, b1, A1d, A1s)


def _tc2_body(acc_ref, den_ref, w_ref, b_ref, a2d_ref, a2s_ref,
              h2_ref, ad_ref, as_ref):
  j = pl.program_id(1)
  den = (den_ref[0] + den_ref[1]) * 0.5
  cmask = lax.broadcasted_iota(jnp.int32, (_RB, 128), 1) == (j // 2)
  dcol = jnp.sum(jnp.where(cmask, den, 0.0), axis=1, keepdims=True)
  hin = (acc_ref[0, 0] + acc_ref[0, 1]) / (dcol + 1e-16)
  hin = jnp.where(hin > 0, hin, jnp.exp(hin) - 1.0)
  part = jnp.dot(hin, w_ref[...], preferred_element_type=jnp.float32)

  @pl.when(j == 0)
  def _init():
    h2_ref[...] = part + b_ref[...]

  @pl.when(j > 0)
  def _acc():
    h2_ref[...] += part

  @pl.when(j == 15)
  def _fin():
    h2 = h2_ref[...]
    ad_ref[...] = jnp.dot(h2, a2d_ref[...], preferred_element_type=jnp.float32)
    as_ref[...] = jnp.dot(h2, a2s_ref[...], preferred_element_type=jnp.float32)


def _tc2(acc1, den1, W2, b2, A2d, A2s):
  return pl.pallas_call(
      _tc2_body,
      grid=(_NRB, 16),
      in_specs=[
          pl.BlockSpec((1, 2, _RB, 128), lambda i, j: (j, 0, i, 0)),
          pl.BlockSpec((2, _RB, 128), lambda i, j: (0, i, 0)),
          pl.BlockSpec((128, _HID), lambda i, j: (j, 0)),
          pl.BlockSpec((1, _HID), lambda i, j: (0, 0)),
          pl.BlockSpec((_HID, 128), lambda i, j: (0, 0)),
          pl.BlockSpec((_HID, 128), lambda i, j: (0, 0)),
      ],
      out_specs=[
          pl.BlockSpec((_RB, _HID), lambda i, j: (i, 0)),
          pl.BlockSpec((_RB, 128), lambda i, j: (i, 0)),
          pl.BlockSpec((_RB, 128), lambda i, j: (i, 0)),
      ],
      out_shape=[
          jax.ShapeDtypeStruct((_N, _HID), jnp.float32),
          jax.ShapeDtypeStruct((_N, 128), jnp.float32),
          jax.ShapeDtypeStruct((_N, 128), jnp.float32),
      ],
  )(acc1, den1, W2, b2, A2d, A2s)


def _tc3_body(acc_ref, den_ref, batch_ref, wfc_ref, bfc_ref, out_ref,
              pooled_ref, cnt_ref):
  i = pl.program_id(0)
  j = pl.program_id(1)
  den = (den_ref[0] + den_ref[1]) * 0.5
  h = (acc_ref[0, 0] + acc_ref[0, 1]) / (den[:, 0:1] + 1e-16)
  h = jnp.where(h > 0, h, jnp.exp(h) - 1.0)
  b = batch_ref[...].reshape(1, _RB)
  gids = lax.broadcasted_iota(jnp.int32, (_NGRAPH, _RB), 0)
  onehot = jnp.where(gids == b, 1.0, 0.0).astype(jnp.float32)
  v = jnp.dot(onehot, h, preferred_element_type=jnp.float32)

  @pl.when(i == 0)
  def _init():
    pooled_ref[pl.ds(j, 1)] = v.reshape(1, _NGRAPH, 128)

  @pl.when(i > 0)
  def _acc():
    pooled_ref[pl.ds(j, 1)] += v.reshape(1, _NGRAPH, 128)

  @pl.when(j == 0)
  def _cnt():
    c = jnp.broadcast_to(
        jnp.sum(onehot, axis=1, keepdims=True), (_NGRAPH, 128))

    @pl.when(i == 0)
    def _ci():
      cnt_ref[...] = c

    @pl.when(i > 0)
    def _ca():
      cnt_ref[...] += c

  @pl.when((i == _NRB - 1) & (j == 1))
  def _fin():
    pm = jnp.concatenate([pooled_ref[0], pooled_ref[1]], axis=1)
    cnt = jnp.maximum(cnt_ref[...][:, 0:1], 1.0)
    pm = pm / cnt
    out_ref[...] = (
        jnp.dot(pm, wfc_ref[...], preferred_element_type=jnp.float32)
        + bfc_ref[...])


def _tc3(acc2, den2, batch3d, Wfc_p, bfc_p):
  return pl.pallas_call(
      _tc3_body,
      grid=(_NRB, 2),
      in_specs=[
          pl.BlockSpec((1, 2, _RB, 128), lambda i, j: (j, 0, i, 0)),
          pl.BlockSpec((2, _RB, 128), lambda i, j: (0, i, 0)),
          pl.BlockSpec((1, 1, _RB), lambda i, j: (i, 0, 0)),
          pl.BlockSpec((_HID, 128), lambda i, j: (0, 0)),
          pl.BlockSpec((1, 128), lambda i, j: (0, 0)),
      ],
      out_specs=pl.BlockSpec((_NGRAPH, 128), lambda i, j: (0, 0)),
      out_shape=jax.ShapeDtypeStruct((_NGRAPH, 128), jnp.float32),
      scratch_shapes=[
          pltpu.VMEM((2, _NGRAPH, 128), jnp.float32),
          pltpu.VMEM((_NGRAPH, 128), jnp.float32),
      ],
  )(acc2, den2, batch3d, Wfc_p, bfc_p)


_ROWS_PT = _NP // 16
_G = 48
_NGRP = _EPT // _G
_GC = 8
_NCH = _NGRP // _GC


def _make_edge_kernel(layer):
  pps = 16 if layer == 1 else 2
  row_w = 128 if layer == 1 else 256
  halfc = _NCH // 2
  mesh = plsc.VectorSubcoreMesh(core_axis_name="c", subcore_axis_name="s")

  @functools.partial(
      pl.kernel,
      out_type=[
          jax.ShapeDtypeStruct((pps, 2, _NP, 128), jnp.float32),
          jax.ShapeDtypeStruct((2, _NP, 128), jnp.float32),
          jax.ShapeDtypeStruct((2, _E_PAD, 128), jnp.float32),
      ],
      mesh=mesh,
      scratch_types=[
          pltpu.VMEM((_GC, _G), jnp.int32),
          pltpu.VMEM((_GC, _G), jnp.int32),
          pltpu.VMEM((_G, row_w), jnp.float32),
          pltpu.VMEM((_G, 128), jnp.float32),
          pltpu.VMEM((_G, 128), jnp.float32),
          pltpu.VMEM((_G, 128), jnp.float32),
          pltpu.VMEM((8, 128), jnp.float32),
          pltpu.VMEM_SHARED((_NP, 128), jnp.float32),
          pltpu.SemaphoreType.DMA,
      ],
  )
  def edge_kernel(src3_hbm, dst3_hbm, h_hbm, adw_hbm, asw_hbm,
                  acc_hbm, den_hbm, w_hbm,
                  src_v, dst_v, rows_v, prod_v, adw_v, asw_v, zb_v,
                  acc_sh, sem):
    core = lax.axis_index("c")
    sid = lax.axis_index("s")

    def _z(r, _):
      def _zc(c, _2):
        zb_v[r, pl.ds(c * 16, 16)] = jnp.zeros((16,), jnp.float32)
        return 0
      lax.fori_loop(0, 8, _zc, 0)
      return 0
    lax.fori_loop(0, 8, _z, 0)

    def _zero_panel():
      def _za(kk, _):
        pltpu.sync_copy(zb_v, acc_sh.at[pl.ds(sid * _ROWS_PT + kk * 8, 8)])
        return 0
      lax.fori_loop(0, _ROWS_PT // 8, _za, 0)

    def _stage(c2):
      pltpu.sync_copy(src3_hbm.at[sid].at[pl.ds(c2 * _GC, _GC)], src_v)
      pltpu.sync_copy(dst3_hbm.at[sid].at[pl.ds(c2 * _GC, _GC)], dst_v)

    def _zp(r, _):
      def _zc2(c, _2):
        prod_v[r, pl.ds(c * 16, 16)] = jnp.zeros((16,), jnp.float32)
        return 0
      lax.fori_loop(0, 8, _zc2, 0)
      return 0
    lax.fori_loop(0, _G, _zp, 0)

    _zero_panel()
    plsc.subcore_barrier()

    def den_chunk(c2, _):
      _stage(c2)

      def den_group(g, _2):
        sidx = src_v.at[g]
        didx = dst_v.at[g]
        c2a = pltpu.async_copy(adw_hbm.at[didx], adw_v, sem)
        c2b = pltpu.async_copy(asw_hbm.at[sidx], asw_v, sem)
        c2a.wait()
        c2b.wait()

        def wrow(r, _3):
          e = adw_v[r, pl.ds(0, 16)] + asw_v[r, pl.ds(0, 16)]
          e = jnp.where(e > 0, e, 0.2 * e)
          prod_v[r, pl.ds(0, 16)] = jnp.exp(e)
          return 0
        lax.fori_loop(0, _G, wrow, 0)
        pltpu.sync_copy(
            prod_v,
            w_hbm.at[core].at[pl.ds(sid * _EPT + (c2 * _GC + g) * _G, _G)])
        pltpu.sync_copy(prod_v, acc_sh.at[didx], add=True)
        return 0
      lax.fori_loop(0, _GC, den_group, 0)
      return 0
    lax.fori_loop(0, _NCH, den_chunk, 0)

    plsc.subcore_barrier()
    pltpu.sync_copy(
        acc_sh.at[pl.ds(sid * _ROWS_PT, _ROWS_PT)],
        den_hbm.at[core].at[pl.ds(sid * _ROWS_PT, _ROWS_PT)])
    plsc.subcore_barrier()

    for p in range(pps):
      head = p // 2 if layer == 1 else 0
      col0 = 0 if layer == 1 else (p % 2) * 128

      _zero_panel()
      plsc.subcore_barrier()

      def do_chunk(c2, _):
        _stage(c2)

        def do_group(g, _2, c2=c2):
          sidx = src_v.at[g]
          didx = dst_v.at[g]
          if layer == 1:
            c1 = pltpu.async_copy(h_hbm.at[p].at[sidx], rows_v, sem)
          else:
            c1 = pltpu.async_copy(h_hbm.at[sidx], rows_v, sem)
          c2a = pltpu.async_copy(
              w_hbm.at[core].at[
                  pl.ds(sid * _EPT + (c2 * _GC + g) * _G, _G)],
              adw_v, sem)
          c1.wait()
          c2a.wait()

          def mrow(r, _3):
            w16 = adw_v[r, pl.ds(0, 16)]
            wsp = jnp.full((16,), w16[head], jnp.float32)
            for c in range(8):
              prod_v[r, pl.ds(c * 16, 16)] = (
                  rows_v[r, pl.ds(col0 + c * 16, 16)] * wsp)
            return 0
          lax.fori_loop(0, _G, mrow, 0)
          pltpu.sync_copy(prod_v, acc_sh.at[didx], add=True)
          return 0
        lax.fori_loop(0, _GC, do_group, 0)
        return 0
      lax.fori_loop(core * halfc, (core + 1) * halfc, do_chunk, 0)

      plsc.subcore_barrier()
      pltpu.sync_copy(
          acc_sh.at[pl.ds(sid * _ROWS_PT, _ROWS_PT)],
          acc_hbm.at[p].at[core].at[pl.ds(sid * _ROWS_PT, _ROWS_PT)])
      plsc.subcore_barrier()

  return edge_kernel


_edge_l1 = _make_edge_kernel(1)
_edge_l2 = _make_edge_kernel(2)


def _pad_rows(a, rows):
  return jnp.concatenate(
      [a, jnp.zeros((rows - a.shape[0],) + a.shape[1:], a.dtype)], axis=0)


def _score_tables(ad, asr):
  return _pad_rows(ad, _NT), _pad_rows(asr, _NT)


@jax.jit
def kernel(x, edge_index, batch, W1, b1, att1, W2, b2, att2, Wfc, bfc):
  loops = jnp.arange(_N, dtype=edge_index.dtype)
  src = jnp.concatenate(
      [edge_index[0], loops,
       jnp.zeros((_E_PAD - _E_FULL,), edge_index.dtype)])
  dst = jnp.concatenate(
      [edge_index[1], loops,
       jnp.full((_E_PAD - _E_FULL,), _N, edge_index.dtype)])
  src3 = src.reshape(16, _NGRP, _G)
  dst3 = dst.reshape(16, _NGRP, _G)

  def att_mats(att, heads):
    ad = att[:, :_HID]
    asr = att[:, _HID:]

    def expand(a):
      m = jnp.zeros((heads * _HID, 128), jnp.float32)
      for h in range(heads):
        m = m.at[h * _HID:(h + 1) * _HID, h].set(a[h])
      return m
    return expand(ad), expand(asr)

  A1d, A1s = att_mats(att1, _HEADS)
  A2d, A2s = att_mats(att2, 1)

  hp1, ad1, as1 = _tc1(x, W1, b1.reshape(1, -1), A1d, A1s)
  adw1, asw1 = _score_tables(ad1, as1)

  acc1, den1, _w1 = _edge_l1(src3, dst3, hp1, adw1, asw1)

  h2, ad2, as2 = _tc2(acc1, den1, W2, b2.reshape(1, -1), A2d, A2s)
  adw2, asw2 = _score_tables(ad2, as2)

  acc2, den2, _w2 = _edge_l2(src3, dst3, h2, adw2, asw2)

  batch3d = batch.reshape(_NRB, 1, _RB)
  Wfc_p = jnp.concatenate(
      [Wfc, jnp.zeros((_HID, 127), jnp.float32)], axis=1)
  bfc_p = jnp.concatenate(
      [bfc.reshape(1, 1), jnp.zeros((1, 127), jnp.float32)], axis=1)
  res = _tc3(acc2, den2, batch3d, Wfc_p, bfc_p)
  return res[:, :1]
